# baseline (speedup 1.0000x reference)
import functools

import jax
import jax.numpy as jnp
from jax import lax
from jax.experimental import pallas as pl
from jax.experimental.pallas import tpu as pltpu
from jax.experimental.pallas import tpu_sc as plsc

R = 128
N = 8192
L = 16
NC = 2
NS = 16
NW = NC * NS
RPW = R // NW
CHUNKS = N // L

_mesh = plsc.VectorSubcoreMesh(core_axis_name="c", subcore_axis_name="s")


@functools.partial(
    pl.kernel,
    out_type=jax.ShapeDtypeStruct((R, L), jnp.int32),
    mesh=_mesh,
    compiler_params=pltpu.CompilerParams(needs_layout_passes=False),
    scratch_types=[
        pltpu.VMEM((2, N), jnp.float32),
        pltpu.VMEM((L,), jnp.int32),
        pltpu.SemaphoreType.DMA,
        pltpu.SemaphoreType.DMA,
    ],
)
def _argmin_sc(x_hbm, out_hbm, buf, res_v, sem0, sem1):
    wid = lax.axis_index("s") * NC + lax.axis_index("c")
    base = wid * RPW
    sems = (sem0, sem1)

    copies = [None] * RPW
    copies[0] = pltpu.async_copy(x_hbm.at[base], buf.at[0], sems[0])
    for r in range(RPW):
        cur = r % 2
        if r + 1 < RPW:
            copies[r + 1] = pltpu.async_copy(
                x_hbm.at[base + r + 1], buf.at[(r + 1) % 2], sems[(r + 1) % 2]
            )
        copies[r].wait()

        iota = lax.iota(jnp.int32, L)
        init = (
            jnp.full((L,), jnp.inf, jnp.float32),
            jnp.zeros((L,), jnp.int32),
            iota,
        )

        def body(i, carry):
            mv, mi, iv = carry
            chunk = buf[cur, pl.ds(i * L, L)]
            pred = chunk < mv
            mv = jnp.where(pred, chunk, mv)
            mi = jnp.where(pred, iv, mi)
            return mv, mi, iv + L

        mv, mi, _ = lax.fori_loop(0, CHUNKS, body, init)

        m = jnp.min(mv)
        cand = jnp.where(mv == m, mi, jnp.int32(2**31 - 1))
        res = jnp.min(cand)
        res_v[...] = jnp.full((L,), res, jnp.int32)
        pltpu.sync_copy(res_v, out_hbm.at[base + r])


def kernel(x):
    return _argmin_sc(x)[:, 0]


# speedup vs baseline: 1.2086x; 1.2086x over previous
import functools

import jax
import jax.numpy as jnp
from jax import lax
from jax.experimental import pallas as pl
from jax.experimental.pallas import tpu as pltpu
from jax.experimental.pallas import tpu_sc as plsc

R = 128
N = 8192
L = 16
NC = 2
NS = 16
NW = NC * NS
RPW = R // NW
U = 8
T = N // (L * U)

_mesh = plsc.VectorSubcoreMesh(core_axis_name="c", subcore_axis_name="s")


@functools.partial(
    pl.kernel,
    out_type=jax.ShapeDtypeStruct((R, L), jnp.int32),
    mesh=_mesh,
    compiler_params=pltpu.CompilerParams(needs_layout_passes=False),
    scratch_types=[
        pltpu.VMEM((2, N), jnp.float32),
        pltpu.VMEM((RPW, L), jnp.int32),
        pltpu.SemaphoreType.DMA,
        pltpu.SemaphoreType.DMA,
    ],
)
def _argmin_sc(x_hbm, out_hbm, buf, res_v, sem0, sem1):
    wid = lax.axis_index("s") * NC + lax.axis_index("c")
    base = wid * RPW
    sems = (sem0, sem1)

    copies = [None] * RPW
    copies[0] = pltpu.async_copy(x_hbm.at[base], buf.at[0], sems[0])
    for r in range(RPW):
        cur = r % 2
        if r + 1 < RPW:
            copies[r + 1] = pltpu.async_copy(
                x_hbm.at[base + r + 1], buf.at[(r + 1) % 2], sems[(r + 1) % 2]
            )
        copies[r].wait()

        inf = jnp.full((L,), jnp.inf, jnp.float32)
        zero = jnp.zeros((L,), jnp.int32)
        init = (inf,) * U + (zero,) * U

        def body(t, carry):
            mvs = list(carry[:U])
            tvs = list(carry[U:])
            tvec = jnp.full((L,), t, jnp.int32)
            for j in range(U):
                chunk = buf[cur, pl.ds((t * U + j) * L, L)]
                pred = chunk < mvs[j]
                mvs[j] = jnp.where(pred, chunk, mvs[j])
                tvs[j] = jnp.where(pred, tvec, tvs[j])
            return tuple(mvs) + tuple(tvs)

        out = lax.fori_loop(0, T, body, init)
        mvs = list(out[:U])
        tvs = list(out[U:])

        iota = lax.iota(jnp.int32, L)
        cols = [(tvs[j] * U + j) * L + iota for j in range(U)]

        step = 1
        while step < U:
            for j in range(0, U, 2 * step):
                a, b = j, j + step
                take_b = (mvs[b] < mvs[a]) | (
                    (mvs[b] == mvs[a]) & (cols[b] < cols[a])
                )
                mvs[a] = jnp.where(take_b, mvs[b], mvs[a])
                cols[a] = jnp.where(take_b, cols[b], cols[a])
            step *= 2

        m = jnp.min(mvs[0])
        cand = jnp.where(mvs[0] == m, cols[0], jnp.int32(2**31 - 1))
        res = jnp.min(cand)
        res_v[r] = jnp.full((L,), res, jnp.int32)

    pltpu.sync_copy(res_v, out_hbm.at[pl.ds(base, RPW)])


def kernel(x):
    return _argmin_sc(x)[:, 0]


# speedup vs baseline: 1.2863x; 1.0643x over previous
import functools

import jax
import jax.numpy as jnp
from jax import lax
from jax.experimental import pallas as pl
from jax.experimental.pallas import tpu as pltpu
from jax.experimental.pallas import tpu_sc as plsc

R = 128
N = 8192
L = 16
NC = 2
NS = 16
NW = NC * NS
RPW = R // NW
U = 8
T = N // (L * U)

_mesh = plsc.VectorSubcoreMesh(core_axis_name="c", subcore_axis_name="s")


@functools.partial(
    pl.kernel,
    out_type=jax.ShapeDtypeStruct((R,), jnp.int32),
    mesh=_mesh,
    compiler_params=pltpu.CompilerParams(
        needs_layout_passes=False,
        disable_bounds_checks=True,
        disable_semaphore_checks=True,
    ),
    scratch_types=[
        pltpu.VMEM((2, N), jnp.float32),
        pltpu.VMEM((L,), jnp.int32),
        pltpu.VMEM((RPW, L), jnp.int32),
        pltpu.VMEM((L,), jnp.int32),
        pltpu.VMEM_SHARED((NS, L), jnp.int32),
        pltpu.SemaphoreType.DMA,
        pltpu.SemaphoreType.DMA,
    ],
)
def _argmin_sc(x_hbm, out_hbm, buf, res_v, gbuf, cbuf, shared, sem0, sem1):
    cid = lax.axis_index("c")
    sid = lax.axis_index("s")
    base = (cid * NS + sid) * RPW
    sems = (sem0, sem1)
    iota = lax.iota(jnp.int32, L)

    resvec = jnp.zeros((L,), jnp.int32)
    copies = [None] * RPW
    copies[0] = pltpu.async_copy(x_hbm.at[base], buf.at[0], sems[0])
    for r in range(RPW):
        cur = r % 2
        if r + 1 < RPW:
            copies[r + 1] = pltpu.async_copy(
                x_hbm.at[base + r + 1], buf.at[(r + 1) % 2], sems[(r + 1) % 2]
            )
        copies[r].wait()

        inf = jnp.full((L,), jnp.inf, jnp.float32)
        zero = jnp.zeros((L,), jnp.int32)
        init = (inf,) * U + (zero,) * U

        def body(t, carry):
            mvs = list(carry[:U])
            tvs = list(carry[U:])
            tvec = jnp.full((L,), t, jnp.int32)
            for j in range(U):
                chunk = buf[cur, pl.ds((t * U + j) * L, L)]
                pred = chunk < mvs[j]
                mvs[j] = jnp.where(pred, chunk, mvs[j])
                tvs[j] = jnp.where(pred, tvec, tvs[j])
            return tuple(mvs) + tuple(tvs)

        out = lax.fori_loop(0, T, body, init)
        mvs = list(out[:U])
        tvs = list(out[U:])

        cols = [(tvs[j] * U + j) * L + iota for j in range(U)]

        step = 1
        while step < U:
            for j in range(0, U, 2 * step):
                a, b = j, j + step
                take_b = (mvs[b] < mvs[a]) | (
                    (mvs[b] == mvs[a]) & (cols[b] < cols[a])
                )
                mvs[a] = jnp.where(take_b, mvs[b], mvs[a])
                cols[a] = jnp.where(take_b, cols[b], cols[a])
            step *= 2

        m = jnp.min(mvs[0])
        cand = jnp.where(mvs[0] == m, cols[0], jnp.int32(2**31 - 1))
        res = jnp.min(cand)
        resvec = jnp.where(iota == r, jnp.full((L,), res, jnp.int32), resvec)

    res_v[...] = resvec
    pltpu.sync_copy(res_v, shared.at[sid])
    plsc.subcore_barrier()

    @pl.when(sid % 4 == 0)
    def _write():
        pltpu.sync_copy(shared.at[pl.ds(sid, 4)], gbuf)
        compact = plsc.load_gather(
            gbuf, [jax.lax.shift_right_logical(iota, 2), iota & 3]
        )
        cbuf[...] = compact
        off = pl.multiple_of((cid * NS + sid) * RPW, L)
        pltpu.sync_copy(cbuf, out_hbm.at[pl.ds(off, L)])


def kernel(x):
    return _argmin_sc(x)
